# baseline (speedup 1.0000x reference)
import jax
import jax.numpy as jnp
from jax import lax
from jax.experimental import pallas as pl
from jax.experimental.pallas import tpu as pltpu
from jax.experimental.pallas import tpu_sc as plsc

_D = 32
_C = 20
_NC, _NS = 2, 16
_NW = _NC * _NS
_SZ = 800
_TBS = 2048


def _tc_transpose_body(x_ref, o_ref):
    o_ref[...] = x_ref[...].T


def _transpose_tc(x2):
    n, c = x2.shape
    return pl.pallas_call(
        _tc_transpose_body,
        grid=(n // _TBS,),
        in_specs=[pl.BlockSpec((_TBS, c), lambda i: (i, 0))],
        out_specs=pl.BlockSpec((c, _TBS), lambda i: (0, i)),
        out_shape=jax.ShapeDtypeStruct((c, n), jnp.int32),
    )(x2)


def _sc_body(xt_hbm, table_hbm, out_hbm, idx_v, acc_v, isem, gsem, g0sem, osem):
    wid = lax.axis_index("s") * _NC + lax.axis_index("c")
    n_rows = out_hbm.shape[0]
    per_w = n_rows // _NW
    chunks = per_w // _SZ

    def fire_idx(i):
        return pltpu.async_copy(
            xt_hbm.at[:, pl.ds(wid * per_w + i * _SZ, _SZ)], idx_v.at[i % 2],
            isem.at[i % 2],
        )

    def fire_out(i):
        return pltpu.async_copy(
            acc_v.at[i % 2], out_hbm.at[pl.ds(wid * per_w + i * _SZ, _SZ)],
            osem.at[i % 2],
        )

    idx_cp = [None] * chunks
    out_cp = [None] * chunks
    adds_prev = None
    idx_cp[0] = fire_idx(0)
    for i in range(chunks):
        b = i % 2
        if i >= 2:
            out_cp[i - 2].wait()
        idx_cp[i].wait()
        g0 = pltpu.async_copy(table_hbm.at[idx_v.at[b].at[0]], acc_v.at[b],
                              g0sem.at[b])
        if adds_prev is not None:
            for cp in adds_prev:
                cp.wait()
            out_cp[i - 1] = fire_out(i - 1)
        if i + 1 < chunks:
            idx_cp[i + 1] = fire_idx(i + 1)
        g0.wait()
        adds_prev = [
            pltpu.async_copy(table_hbm.at[idx_v.at[b].at[c]], acc_v.at[b],
                             gsem.at[b], add=True)
            for c in range(1, _C)
        ]
    for cp in adds_prev:
        cp.wait()
    out_cp[chunks - 1] = fire_out(chunks - 1)
    out_cp[chunks - 2].wait()
    out_cp[chunks - 1].wait()


def kernel(x, table):
    b, v, c = x.shape
    n = b * v
    xt = _transpose_tc(x.reshape(n, c))
    run = pl.kernel(
        _sc_body,
        out_type=jax.ShapeDtypeStruct((n, _D), jnp.float32),
        mesh=plsc.VectorSubcoreMesh(core_axis_name="c", subcore_axis_name="s"),
        scratch_types=[
            pltpu.VMEM((2, _C, _SZ), jnp.int32),
            pltpu.VMEM((2, _SZ, _D), jnp.float32),
            pltpu.SemaphoreType.DMA((2,)),
            pltpu.SemaphoreType.DMA((2,)),
            pltpu.SemaphoreType.DMA((2,)),
            pltpu.SemaphoreType.DMA((2,)),
        ],
        compiler_params=pltpu.CompilerParams(use_tc_tiling_on_sc=False),
    )
    out = run(xt, table)
    return out.reshape(b, v, _D)


# speedup vs baseline: 1.2184x; 1.2184x over previous
import jax
import jax.numpy as jnp
from jax import lax
from jax.experimental import pallas as pl
from jax.experimental.pallas import tpu as pltpu
from jax.experimental.pallas import tpu_sc as plsc

_D = 32
_C = 20
_NC, _NS = 2, 16
_NW = _NC * _NS
_SZ = 800
_TBS = 2048


def _tc_transpose_body(x_ref, o_ref):
    o_ref[...] = x_ref[...].T


def _transpose_tc(x2):
    n, c = x2.shape
    return pl.pallas_call(
        _tc_transpose_body,
        grid=(n // _TBS,),
        in_specs=[pl.BlockSpec((_TBS, c), lambda i: (i, 0))],
        out_specs=pl.BlockSpec((c, _TBS), lambda i: (0, i)),
        out_shape=jax.ShapeDtypeStruct((c, n), jnp.int32),
    )(x2)


def _sc_body(xt_hbm, table_hbm, out_hbm, idx_v, acc_v, isem, gsem, g0sem, osem):
    wid = lax.axis_index("s") * _NC + lax.axis_index("c")
    n_rows = out_hbm.shape[0]
    per_w = n_rows // _NW
    chunks = per_w // _SZ

    def fire_idx(i):
        return pltpu.async_copy(
            xt_hbm.at[pl.ds(0, _C), pl.ds(wid * per_w + i * _SZ, _SZ)],
            idx_v.at[i % 2], isem.at[i % 2],
        )

    def fire_out(i):
        return pltpu.async_copy(
            acc_v.at[i % 2], out_hbm.at[pl.ds(wid * per_w + i * _SZ, _SZ)],
            osem.at[i % 2],
        )

    idx_cp = [None] * chunks
    out_cp = [None] * chunks
    adds_prev = None
    idx_cp[0] = fire_idx(0)
    for i in range(chunks):
        b = i % 2
        if i >= 2:
            out_cp[i - 2].wait()
        idx_cp[i].wait()
        g0 = pltpu.async_copy(table_hbm.at[idx_v.at[b].at[0]], acc_v.at[b],
                              g0sem.at[b])
        if adds_prev is not None:
            for cp in adds_prev:
                cp.wait()
            out_cp[i - 1] = fire_out(i - 1)
        if i + 1 < chunks:
            idx_cp[i + 1] = fire_idx(i + 1)
        g0.wait()
        adds_prev = [
            pltpu.async_copy(table_hbm.at[idx_v.at[b].at[c]], acc_v.at[b],
                             gsem.at[b], add=True)
            for c in range(1, _C)
        ]
    for cp in adds_prev:
        cp.wait()
    out_cp[chunks - 1] = fire_out(chunks - 1)
    out_cp[chunks - 2].wait()
    out_cp[chunks - 1].wait()


def kernel(x, table):
    b, v, c = x.shape
    n = b * v
    xt = jnp.pad(x.reshape(n, c).T, ((0, 4), (0, 0)))
    run = pl.kernel(
        _sc_body,
        out_type=jax.ShapeDtypeStruct((n, _D), jnp.float32),
        mesh=plsc.VectorSubcoreMesh(core_axis_name="c", subcore_axis_name="s"),
        scratch_types=[
            pltpu.VMEM((2, _C, _SZ), jnp.int32),
            pltpu.VMEM((2, _SZ, _D), jnp.float32),
            pltpu.SemaphoreType.DMA((2,)),
            pltpu.SemaphoreType.DMA((2,)),
            pltpu.SemaphoreType.DMA((2,)),
            pltpu.SemaphoreType.DMA((2,)),
        ],
        compiler_params=pltpu.CompilerParams(use_tc_tiling_on_sc=False),
    )
    out = run(xt, table)
    return out.reshape(b, v, _D)


# speedup vs baseline: 1.8645x; 1.5303x over previous
import jax
import jax.numpy as jnp
from jax import lax
from jax.experimental import pallas as pl
from jax.experimental.pallas import tpu as pltpu
from jax.experimental.pallas import tpu_sc as plsc

_D = 32
_C = 20
_NC, _NS = 2, 16
_NW = _NC * _NS
_SZ = 800
_TBS = 2048


def _tc_transpose_body(x_ref, o_ref):
    o_ref[...] = x_ref[...].T


def _transpose_tc(x2):
    n, c = x2.shape
    return pl.pallas_call(
        _tc_transpose_body,
        grid=(n // _TBS,),
        in_specs=[pl.BlockSpec((_TBS, c), lambda i: (i, 0))],
        out_specs=pl.BlockSpec((c, _TBS), lambda i: (0, i)),
        out_shape=jax.ShapeDtypeStruct((c, n), jnp.int32),
    )(x2)


def _sc_body(xt_hbm, table_hbm, out_hbm, idx_v, acc_v, isem, gsem, g0sem, osem):
    wid = lax.axis_index("s") * _NC + lax.axis_index("c")
    n_rows = xt_hbm.shape[1]
    per_w = n_rows // _NW
    chunks = per_w // _SZ

    bpc = _SZ // 50

    def fire_idx(i):
        return pltpu.async_copy(
            xt_hbm.at[:, pl.ds(wid * per_w + i * _SZ, _SZ)],
            idx_v.at[i % 2], isem.at[i % 2],
        )

    def fire_out(i):
        bbase = (wid * per_w + i * _SZ) // 50
        return [
            pltpu.async_copy(
                acc_v.at[i % 2].at[pl.ds(g * 50, 50)],
                out_hbm.at[bbase + g].at[pl.ds(0, 50), pl.ds(0, _D)],
                osem.at[i % 2],
            )
            for g in range(bpc)
        ]

    idx_cp = [None] * chunks
    out_cp = [None] * chunks
    adds_prev = None
    idx_cp[0] = fire_idx(0)
    for i in range(chunks):
        b = i % 2
        if i >= 2:
            for cp in out_cp[i - 2]:
                cp.wait()
        idx_cp[i].wait()
        g0 = pltpu.async_copy(table_hbm.at[idx_v.at[b].at[0]], acc_v.at[b],
                              g0sem.at[b])
        if adds_prev is not None:
            for cp in adds_prev:
                cp.wait()
            out_cp[i - 1] = fire_out(i - 1)
        if i + 1 < chunks:
            idx_cp[i + 1] = fire_idx(i + 1)
        g0.wait()
        adds_prev = [
            pltpu.async_copy(table_hbm.at[idx_v.at[b].at[c]], acc_v.at[b],
                             gsem.at[b], add=True)
            for c in range(1, _C)
        ]
    for cp in adds_prev:
        cp.wait()
    out_cp[chunks - 1] = fire_out(chunks - 1)
    for cp in out_cp[chunks - 2] + out_cp[chunks - 1]:
        cp.wait()


def kernel(x, table):
    b, v, c = x.shape
    n = b * v
    xt = x.reshape(n, c).T
    run = pl.kernel(
        _sc_body,
        out_type=jax.ShapeDtypeStruct((b, 56, 128), jnp.float32),
        mesh=plsc.VectorSubcoreMesh(core_axis_name="c", subcore_axis_name="s"),
        scratch_types=[
            pltpu.VMEM((2, _C, _SZ), jnp.int32),
            pltpu.VMEM((2, _SZ, _D), jnp.float32),
            pltpu.SemaphoreType.DMA((2,)),
            pltpu.SemaphoreType.DMA((2,)),
            pltpu.SemaphoreType.DMA((2,)),
            pltpu.SemaphoreType.DMA((2,)),
        ],
        compiler_params=pltpu.CompilerParams(use_tc_tiling_on_sc=False),
    )
    out = run(xt, table)
    return out[:, :v, :_D]
